# Initial kernel scaffold; baseline (speedup 1.0000x reference)
import jax
import jax.numpy as jnp
from jax.experimental import pallas as pl


def kernel(x, edge_index, timestamps, time_diffs, unique_edges, timestamp_lists, te_w0, te_b0, te_w, te_b, Wm0, bm0, Wu0, Ws0, bo0, Wm1, bm1, Wu1, Ws1, bo1):
    raise NotImplementedError("write your pallas kernel here")


# speedup vs baseline: 1.6501x; 1.6501x over previous
import functools

import jax
import jax.numpy as jnp
from jax import lax
from jax.experimental import pallas as pl
from jax.experimental.pallas import tpu as pltpu
from jax.experimental.pallas import tpu_sc as plsc

N = 10000
E = 320000
EU = 160000
D = 128
H = 128
TF = 16

NC = 2
NS = 16
NW = NC * NS
LANES = 16

EPW = E // NW
EUPW = EU // NW
CHUNK = 40
FLUSH_TILES = 10
FLUSH_ROWS = 1000
ZROWS = 200

@functools.cache
def _mesh():
    return plsc.VectorSubcoreMesh(
        core_axis_name="c", subcore_axis_name="s",
        num_cores=NC, num_subcores=NS,
    )


def _zero_vmem(buf, nrows):
    zero = jnp.zeros((LANES,), jnp.float32)

    def body(g, _):
        r = g // (H // LANES)
        col = (g % (H // LANES)) * LANES
        buf[r, pl.ds(col, LANES)] = zero
        return 0

    lax.fori_loop(0, nrows * (H // LANES), body, 0)


def _acc_init(zero_v, acc_sh, sid):
    @pl.when(sid < FLUSH_TILES)
    def _():
        _zero_vmem(zero_v, ZROWS)
        for t in range(FLUSH_ROWS // ZROWS):
            pltpu.sync_copy(
                zero_v, acc_sh.at[pl.ds(sid * FLUSH_ROWS + t * ZROWS, ZROWS)]
            )


def _acc_flush(acc_sh, out_hbm, cid, sid):
    @pl.when(sid < FLUSH_TILES)
    def _():
        base = sid * FLUSH_ROWS
        pltpu.sync_copy(
            acc_sh.at[pl.ds(base, FLUSH_ROWS)],
            out_hbm.at[cid, pl.ds(base, FLUSH_ROWS)],
        )


def _sc_message_body(zw_hbm, c_hbm, src_hbm, dst_hbm, out_hbm,
                     idx_v, didx_v, rows_v, c_v, zero_v, acc_sh, sem):
    cid = lax.axis_index("c")
    sid = lax.axis_index("s")
    wid = sid * NC + cid

    _acc_init(zero_v, acc_sh, sid)
    plsc.subcore_barrier()

    def chunk(i, _):
        base = wid * EPW + i * CHUNK
        pltpu.sync_copy(src_hbm.at[pl.ds(base, CHUNK)], idx_v)
        pltpu.sync_copy(dst_hbm.at[pl.ds(base, CHUNK)], didx_v)
        pltpu.async_copy(zw_hbm.at[idx_v], rows_v, sem).wait()
        pltpu.sync_copy(c_hbm.at[pl.ds(base, CHUNK)], c_v)

        def row(r, _):
            for j in range(H // LANES):
                col = j * LANES
                v = rows_v[r, pl.ds(col, LANES)] + c_v[r, pl.ds(col, LANES)]
                rows_v[r, pl.ds(col, LANES)] = jnp.maximum(v, 0.0)
            return 0

        lax.fori_loop(0, CHUNK, row, 0)
        pltpu.sync_copy(rows_v, acc_sh.at[didx_v], add=True)
        return 0

    lax.fori_loop(0, EPW // CHUNK, chunk, 0)
    plsc.subcore_barrier()
    _acc_flush(acc_sh, out_hbm, cid, sid)


def _sc_unique_body(z_hbm, src_hbm, dst_hbm, out_hbm,
                    idx_v, didx_v, rows_v, zero_v, acc_sh, sem):
    cid = lax.axis_index("c")
    sid = lax.axis_index("s")
    wid = sid * NC + cid

    _acc_init(zero_v, acc_sh, sid)
    plsc.subcore_barrier()

    def chunk(i, _):
        base = wid * EUPW + i * CHUNK
        pltpu.sync_copy(src_hbm.at[pl.ds(base, CHUNK)], idx_v)
        pltpu.sync_copy(dst_hbm.at[pl.ds(base, CHUNK)], didx_v)
        pltpu.async_copy(z_hbm.at[idx_v], rows_v, sem).wait()
        pltpu.sync_copy(rows_v, acc_sh.at[didx_v], add=True)
        return 0

    lax.fori_loop(0, EUPW // CHUNK, chunk, 0)
    plsc.subcore_barrier()
    _acc_flush(acc_sh, out_hbm, cid, sid)


@functools.cache
def _sc_message():
    return pl.kernel(
        _sc_message_body,
        out_type=jax.ShapeDtypeStruct((NC, N, H), jnp.float32),
        mesh=_mesh(),
        scratch_types=[
            pltpu.VMEM((CHUNK,), jnp.int32),
            pltpu.VMEM((CHUNK,), jnp.int32),
            pltpu.VMEM((CHUNK, H), jnp.float32),
            pltpu.VMEM((CHUNK, H), jnp.float32),
            pltpu.VMEM((ZROWS, H), jnp.float32),
            pltpu.VMEM_SHARED((N, H), jnp.float32),
            pltpu.SemaphoreType.DMA,
        ],
    )


@functools.cache
def _sc_unique():
    return pl.kernel(
        _sc_unique_body,
        out_type=jax.ShapeDtypeStruct((NC, N, H), jnp.float32),
        mesh=_mesh(),
        scratch_types=[
            pltpu.VMEM((CHUNK,), jnp.int32),
            pltpu.VMEM((CHUNK,), jnp.int32),
            pltpu.VMEM((CHUNK, H), jnp.float32),
            pltpu.VMEM((ZROWS, H), jnp.float32),
            pltpu.VMEM_SHARED((N, H), jnp.float32),
            pltpu.SemaphoreType.DMA,
        ],
    )


_EB = 2000
_NB = 2000


def _edge_const_body(tsl_ref, td_ref, s0_ref, tew_ref, teb_ref,
                     wt0_ref, bm0_ref, wt1_ref, bm1_ref, c0_ref, c1_ref):
    t = tsl_ref[0, 0, :]
    lin = (s0_ref[0, 0] * t + s0_ref[0, 1])[:, None]
    per = jnp.sin(t[:, None] * tew_ref[0, :][None, :] + teb_ref[0, :][None, :])
    feats = jnp.concatenate([lin, per, td_ref[0, 0, :][:, None]], axis=1)
    c0_ref[...] = (
        jnp.dot(feats, wt0_ref[...], preferred_element_type=jnp.float32)
        + bm0_ref[0, :][None, :]
    )
    c1_ref[...] = (
        jnp.dot(feats, wt1_ref[...], preferred_element_type=jnp.float32)
        + bm1_ref[0, :][None, :]
    )


def _edge_const(tsl, td, te_w0, te_b0, te_w, te_b, Wt0, bm0, Wt1, bm1):
    grid = E // _EB
    s0 = jnp.stack([te_w0, te_b0]).reshape(1, 2)
    out = pl.pallas_call(
        _edge_const_body,
        grid=(grid,),
        in_specs=[
            pl.BlockSpec((1, 1, _EB), lambda i: (i, 0, 0)),
            pl.BlockSpec((1, 1, _EB), lambda i: (i, 0, 0)),
            pl.BlockSpec((1, 2), lambda i: (0, 0)),
            pl.BlockSpec((1, TF - 1), lambda i: (0, 0)),
            pl.BlockSpec((1, TF - 1), lambda i: (0, 0)),
            pl.BlockSpec((TF + 1, H), lambda i: (0, 0)),
            pl.BlockSpec((1, H), lambda i: (0, 0)),
            pl.BlockSpec((TF + 1, H), lambda i: (0, 0)),
            pl.BlockSpec((1, H), lambda i: (0, 0)),
        ],
        out_specs=[
            pl.BlockSpec((_EB, H), lambda i: (i, 0)),
            pl.BlockSpec((_EB, H), lambda i: (i, 0)),
        ],
        out_shape=[
            jax.ShapeDtypeStruct((E, H), jnp.float32),
            jax.ShapeDtypeStruct((E, H), jnp.float32),
        ],
    )(tsl.reshape(grid, 1, _EB), td.reshape(grid, 1, _EB), s0,
      te_w.reshape(1, TF - 1), te_b.reshape(1, TF - 1), Wt0,
      bm0.reshape(1, H), Wt1, bm1.reshape(1, H))
    return out


def _matmul_body(x_ref, w_ref, o_ref):
    o_ref[...] = jnp.dot(x_ref[...], w_ref[...],
                         preferred_element_type=jnp.float32)


def _matmul(xm, w):
    return pl.pallas_call(
        _matmul_body,
        grid=(N // _NB,),
        in_specs=[
            pl.BlockSpec((_NB, D), lambda i: (i, 0)),
            pl.BlockSpec((D, H), lambda i: (0, 0)),
        ],
        out_specs=pl.BlockSpec((_NB, H), lambda i: (i, 0)),
        out_shape=jax.ShapeDtypeStruct((N, H), jnp.float32),
    )(xm, w)


def _combine_body(agg_ref, uagg_ref, z_ref, wu_ref, ws_ref, bo_ref, o_ref):
    u = uagg_ref[0] + uagg_ref[1]
    acc = agg_ref[0] + agg_ref[1]
    acc += jnp.dot(u, wu_ref[...], preferred_element_type=jnp.float32)
    acc += jnp.dot(z_ref[...], ws_ref[...], preferred_element_type=jnp.float32)
    o_ref[...] = jnp.maximum(acc + bo_ref[0, :][None, :], 0.0)


def _combine(agg, uagg, z, Wu, Ws, bo):
    return pl.pallas_call(
        _combine_body,
        grid=(N // _NB,),
        in_specs=[
            pl.BlockSpec((NC, _NB, H), lambda i: (0, i, 0)),
            pl.BlockSpec((NC, _NB, H), lambda i: (0, i, 0)),
            pl.BlockSpec((_NB, D), lambda i: (i, 0)),
            pl.BlockSpec((D, H), lambda i: (0, 0)),
            pl.BlockSpec((D, H), lambda i: (0, 0)),
            pl.BlockSpec((1, H), lambda i: (0, 0)),
        ],
        out_specs=pl.BlockSpec((_NB, H), lambda i: (i, 0)),
        out_shape=jax.ShapeDtypeStruct((N, H), jnp.float32),
    )(agg, uagg, z, Wu, Ws, bo.reshape(1, H))


def kernel(x, edge_index, timestamps, time_diffs, unique_edges,
           timestamp_lists, te_w0, te_b0, te_w, te_b,
           Wm0, bm0, Wu0, Ws0, bo0, Wm1, bm1, Wu1, Ws1, bo1):
    src = edge_index[0]
    dst = edge_index[1]
    usrc = unique_edges[0]
    udst = unique_edges[1]

    c0, c1 = _edge_const(timestamp_lists, time_diffs, te_w0, te_b0,
                         te_w, te_b, Wm0[D:], bm0, Wm1[D:], bm1)

    z = x
    for Wm, c, Wu, Ws, bo in ((Wm0, c0, Wu0, Ws0, bo0),
                              (Wm1, c1, Wu1, Ws1, bo1)):
        zw = _matmul(z, Wm[:D])
        agg = _sc_message()(zw, c, src, dst)
        uagg = _sc_unique()(z, usrc, udst)
        z = _combine(agg, uagg, z, Wu, Ws, bo)
    return z


# speedup vs baseline: 3.8021x; 2.3041x over previous
import functools

import jax
import jax.numpy as jnp
from jax import lax
from jax.experimental import pallas as pl
from jax.experimental.pallas import tpu as pltpu
from jax.experimental.pallas import tpu_sc as plsc

N = 10000
E = 320000
EU = 160000
D = 128
H = 128
TF = 16

NC = 2
NS = 16
NW = NC * NS
LANES = 16
LG = H // LANES

EPW = E // NW
EUPW = EU // NW
MC = 40
MCH = EPW // MC
UCH = EUPW // MC
NBUF = 3
NIB = 2 * NBUF

FLUSH_TILES = 10
FLUSH_ROWS = 1000


@functools.cache
def _mesh():
    return plsc.VectorSubcoreMesh(
        core_axis_name="c", subcore_axis_name="s",
        num_cores=NC, num_subcores=NS,
    )


def _sc_layer_kernel_body(zw_hbm, zu_hbm, c_hbm, sdr, usdr,
                          out_hbm, *refs):
    rbufs = refs[0:3]
    cbufs = refs[3:6]
    ibufs = refs[6:12]
    ubufs = refs[12:18]
    acc_sh = refs[18]
    gsems = refs[19:22]
    csems = refs[22:25]
    isems = refs[25:31]
    usems = refs[31:37]

    cid = lax.axis_index("c")
    sid = lax.axis_index("s")
    wid = sid * NC + cid

    @pl.when(sid < FLUSH_TILES)
    def _():
        zero = jnp.zeros((LANES,), jnp.float32)

        def zrow(g, _):
            rbufs[0][g // LG, pl.ds((g % LG) * LANES, LANES)] = zero
            return 0

        lax.fori_loop(0, MC * LG, zrow, 0)
        for t in range(FLUSH_ROWS // MC):
            pltpu.sync_copy(
                rbufs[0], acc_sh.at[pl.ds(sid * FLUSH_ROWS + t * MC, MC)]
            )

    plsc.subcore_barrier()

    def make_pipeline(nch, ibase, idx_hbm, idx_bufs, idx_sems,
                      tab_hbm, with_c):
        def fetch(g, r):
            pltpu.async_copy(
                idx_hbm.at[pl.ds(ibase + g, 1)], idx_bufs[r], idx_sems[r]
            )

        def gather(g, r, b):
            pltpu.make_async_copy(
                idx_hbm.at[pl.ds(ibase, 1)], idx_bufs[r], idx_sems[r]
            ).wait()
            pltpu.async_copy(
                tab_hbm.at[idx_bufs[r].at[0, 0]], rbufs[b], gsems[b]
            )
            if with_c:
                pltpu.async_copy(
                    c_hbm.at[pl.ds(wid * EPW + g * MC, MC)],
                    cbufs[b], csems[b],
                )

        def wait_rows(b):
            pltpu.make_async_copy(
                tab_hbm.at[idx_bufs[0].at[0, 0]], rbufs[b], gsems[b]
            ).wait()
            if with_c:
                pltpu.make_async_copy(
                    c_hbm.at[pl.ds(0, MC)], cbufs[b], csems[b]
                ).wait()

        def scatter(r, b):
            pltpu.sync_copy(
                rbufs[b], acc_sh.at[idx_bufs[r].at[0, 1]], add=True
            )

        return fetch, gather, wait_rows, scatter

    def run_pass(nch, ibase, idx_hbm, idx_bufs, idx_sems, tab_hbm, with_c,
                 compute):
        fetch, gather, wait_rows, scatter = make_pipeline(
            nch, ibase, idx_hbm, idx_bufs, idx_sems, tab_hbm, with_c
        )

        for g in range(min(NIB, nch)):
            fetch(g, g % NIB)
        for g in range(min(NBUF, nch)):
            gather(g, g % NIB, g % NBUF)

        def step(g, r, b, r_nxt, guard):
            wait_rows(b)
            if compute is not None:
                compute(b)
            scatter(r, b)
            nxt = g + NBUF
            nxt2 = g + NIB

            def advance():
                gather(nxt, r_nxt, b)

            def refetch():
                fetch(nxt2, r)

            if guard:
                if nxt < nch:
                    advance()
                if nxt2 < nch:
                    refetch()
            else:
                @pl.when(nxt < nch)
                def _():
                    advance()

                @pl.when(nxt2 < nch)
                def _():
                    refetch()

        def body(s, _):
            for k in range(NIB):
                g = s * NIB + k
                step(g, k, k % NBUF, (k + NBUF) % NIB, guard=False)
            return 0

        lax.fori_loop(0, nch // NIB, body, 0)
        for g in range(nch - nch % NIB, nch):
            step(g, g % NIB, g % NBUF, (g + NBUF) % NIB, guard=True)

    def compute_a(b):
        def row(r, _):
            for j in range(LG):
                col = j * LANES
                v = rbufs[b][r, pl.ds(col, LANES)] \
                    + cbufs[b][r, pl.ds(col, LANES)]
                rbufs[b][r, pl.ds(col, LANES)] = jnp.maximum(v, 0.0)
            return 0

        lax.fori_loop(0, MC, row, 0)

    run_pass(MCH, wid * MCH, sdr, ibufs, isems, zw_hbm, True, compute_a)

    run_pass(UCH, wid * UCH, usdr, ubufs, usems, zu_hbm, False, None)

    plsc.subcore_barrier()

    @pl.when(sid < FLUSH_TILES)
    def _():
        base = sid * FLUSH_ROWS
        pltpu.sync_copy(
            acc_sh.at[pl.ds(base, FLUSH_ROWS)],
            out_hbm.at[cid, pl.ds(base, FLUSH_ROWS)],
        )


@functools.cache
def _sc_layer():
    sems = [pltpu.SemaphoreType.DMA] * (3 + 3 + NIB + NIB)
    return pl.kernel(
        _sc_layer_kernel_body,
        out_type=jax.ShapeDtypeStruct((NC, N, H), jnp.float32),
        mesh=_mesh(),
        scratch_types=(
            [pltpu.VMEM((MC, H), jnp.float32)] * 3
            + [pltpu.VMEM((MC, H), jnp.float32)] * 3
            + [pltpu.VMEM((1, 2, MC), jnp.int32)] * NIB
            + [pltpu.VMEM((1, 2, MC), jnp.int32)] * NIB
            + [pltpu.VMEM_SHARED((N, H), jnp.float32)]
            + sems
        ),
    )


_EB = 2000
_NB = 2000


def _edge_const_body(tsl_ref, td_ref, s0_ref, tew_ref, teb_ref,
                     wt0_ref, bm0_ref, wt1_ref, bm1_ref, c0_ref, c1_ref):
    t = tsl_ref[0, 0, :]
    lin = (s0_ref[0, 0] * t + s0_ref[0, 1])[:, None]
    per = jnp.sin(t[:, None] * tew_ref[0, :][None, :] + teb_ref[0, :][None, :])
    feats = jnp.concatenate([lin, per, td_ref[0, 0, :][:, None]], axis=1)
    c0_ref[...] = (
        jnp.dot(feats, wt0_ref[...], preferred_element_type=jnp.float32)
        + bm0_ref[0, :][None, :]
    )
    c1_ref[...] = (
        jnp.dot(feats, wt1_ref[...], preferred_element_type=jnp.float32)
        + bm1_ref[0, :][None, :]
    )


def _edge_const(tsl, td, te_w0, te_b0, te_w, te_b, Wt0, bm0, Wt1, bm1):
    grid = E // _EB
    s0 = jnp.stack([te_w0, te_b0]).reshape(1, 2)
    return pl.pallas_call(
        _edge_const_body,
        grid=(grid,),
        in_specs=[
            pl.BlockSpec((1, 1, _EB), lambda i: (i, 0, 0)),
            pl.BlockSpec((1, 1, _EB), lambda i: (i, 0, 0)),
            pl.BlockSpec((1, 2), lambda i: (0, 0)),
            pl.BlockSpec((1, TF - 1), lambda i: (0, 0)),
            pl.BlockSpec((1, TF - 1), lambda i: (0, 0)),
            pl.BlockSpec((TF + 1, H), lambda i: (0, 0)),
            pl.BlockSpec((1, H), lambda i: (0, 0)),
            pl.BlockSpec((TF + 1, H), lambda i: (0, 0)),
            pl.BlockSpec((1, H), lambda i: (0, 0)),
        ],
        out_specs=[
            pl.BlockSpec((_EB, H), lambda i: (i, 0)),
            pl.BlockSpec((_EB, H), lambda i: (i, 0)),
        ],
        out_shape=[
            jax.ShapeDtypeStruct((E, H), jnp.float32),
            jax.ShapeDtypeStruct((E, H), jnp.float32),
        ],
    )(tsl.reshape(grid, 1, _EB), td.reshape(grid, 1, _EB), s0,
      te_w.reshape(1, TF - 1), te_b.reshape(1, TF - 1), Wt0,
      bm0.reshape(1, H), Wt1, bm1.reshape(1, H))


def _pre_nodes_body(x_ref, wm_ref, wu_ref, zw_ref, zu_ref):
    xb = x_ref[...]
    zw_ref[...] = jnp.dot(xb, wm_ref[...], preferred_element_type=jnp.float32)
    zu_ref[...] = jnp.dot(xb, wu_ref[...], preferred_element_type=jnp.float32)


def _pre_nodes(xm, Wmx, Wu):
    return pl.pallas_call(
        _pre_nodes_body,
        grid=(N // _NB,),
        in_specs=[
            pl.BlockSpec((_NB, D), lambda i: (i, 0)),
            pl.BlockSpec((D, H), lambda i: (0, 0)),
            pl.BlockSpec((D, H), lambda i: (0, 0)),
        ],
        out_specs=[
            pl.BlockSpec((_NB, H), lambda i: (i, 0)),
            pl.BlockSpec((_NB, H), lambda i: (i, 0)),
        ],
        out_shape=[
            jax.ShapeDtypeStruct((N, H), jnp.float32),
            jax.ShapeDtypeStruct((N, H), jnp.float32),
        ],
    )(xm, Wmx, Wu)


def _mid_body(acc_ref, z_ref, ws_ref, bo_ref, wm_ref, wu_ref,
              z1_ref, zw_ref, zu_ref):
    a = acc_ref[0] + acc_ref[1]
    a += jnp.dot(z_ref[...], ws_ref[...], preferred_element_type=jnp.float32)
    z1 = jnp.maximum(a + bo_ref[0, :][None, :], 0.0)
    z1_ref[...] = z1
    zw_ref[...] = jnp.dot(z1, wm_ref[...], preferred_element_type=jnp.float32)
    zu_ref[...] = jnp.dot(z1, wu_ref[...], preferred_element_type=jnp.float32)


def _mid(acc, z, Ws, bo, Wmx, Wu):
    return pl.pallas_call(
        _mid_body,
        grid=(N // _NB,),
        in_specs=[
            pl.BlockSpec((NC, _NB, H), lambda i: (0, i, 0)),
            pl.BlockSpec((_NB, D), lambda i: (i, 0)),
            pl.BlockSpec((D, H), lambda i: (0, 0)),
            pl.BlockSpec((1, H), lambda i: (0, 0)),
            pl.BlockSpec((D, H), lambda i: (0, 0)),
            pl.BlockSpec((D, H), lambda i: (0, 0)),
        ],
        out_specs=[
            pl.BlockSpec((_NB, H), lambda i: (i, 0)),
            pl.BlockSpec((_NB, H), lambda i: (i, 0)),
            pl.BlockSpec((_NB, H), lambda i: (i, 0)),
        ],
        out_shape=[
            jax.ShapeDtypeStruct((N, H), jnp.float32),
            jax.ShapeDtypeStruct((N, H), jnp.float32),
            jax.ShapeDtypeStruct((N, H), jnp.float32),
        ],
    )(acc, z, Ws, bo.reshape(1, H), Wmx, Wu)


def _final_body(acc_ref, z_ref, ws_ref, bo_ref, o_ref):
    a = acc_ref[0] + acc_ref[1]
    a += jnp.dot(z_ref[...], ws_ref[...], preferred_element_type=jnp.float32)
    o_ref[...] = jnp.maximum(a + bo_ref[0, :][None, :], 0.0)


def _final(acc, z, Ws, bo):
    return pl.pallas_call(
        _final_body,
        grid=(N // _NB,),
        in_specs=[
            pl.BlockSpec((NC, _NB, H), lambda i: (0, i, 0)),
            pl.BlockSpec((_NB, D), lambda i: (i, 0)),
            pl.BlockSpec((D, H), lambda i: (0, 0)),
            pl.BlockSpec((1, H), lambda i: (0, 0)),
        ],
        out_specs=pl.BlockSpec((_NB, H), lambda i: (i, 0)),
        out_shape=jax.ShapeDtypeStruct((N, H), jnp.float32),
    )(acc, z, Ws, bo.reshape(1, H))


def kernel(x, edge_index, timestamps, time_diffs, unique_edges,
           timestamp_lists, te_w0, te_b0, te_w, te_b,
           Wm0, bm0, Wu0, Ws0, bo0, Wm1, bm1, Wu1, Ws1, bo1):
    sdr = jnp.stack(
        [edge_index[0].reshape(NW * MCH, MC),
         edge_index[1].reshape(NW * MCH, MC)], axis=1)
    usdr = jnp.stack(
        [unique_edges[0].reshape(NW * UCH, MC),
         unique_edges[1].reshape(NW * UCH, MC)], axis=1)

    c0, c1 = _edge_const(timestamp_lists, time_diffs, te_w0, te_b0,
                         te_w, te_b, Wm0[D:], bm0, Wm1[D:], bm1)

    zw0, zu0 = _pre_nodes(x, Wm0[:D], Wu0)
    acc0 = _sc_layer()(zw0, zu0, c0, sdr, usdr)
    z1, zw1, zu1 = _mid(acc0, x, Ws0, bo0, Wm1[:D], Wu1)
    acc1 = _sc_layer()(zw1, zu1, c1, sdr, usdr)
    return _final(acc1, z1, Ws1, bo1)


# speedup vs baseline: 5.6319x; 1.4813x over previous
import functools

import jax
import jax.numpy as jnp
from jax import lax
from jax.experimental import pallas as pl
from jax.experimental.pallas import tpu as pltpu
from jax.experimental.pallas import tpu_sc as plsc

N = 10000
E = 320000
EU = 160000
D = 128
H = 128
TF = 16

NC = 2
NS = 16
NW = NC * NS
LANES = 16
LG = H // LANES

EPW = E // NW
EUPW = EU // NW
MC = 40
MCH = EPW // MC
UCH = EUPW // MC
NBUF = 3
NIB = 2 * NBUF

FLUSH_TILES = 10
FLUSH_ROWS = 1000


@functools.cache
def _mesh():
    return plsc.VectorSubcoreMesh(
        core_axis_name="c", subcore_axis_name="s",
        num_cores=NC, num_subcores=NS,
    )


def _sc_layer_kernel_body(zw_hbm, zu_hbm, c_hbm, sdr, usdr,
                          out_hbm, *refs):
    rbufs = refs[0:3]
    cbufs = refs[3:6]
    ibufs = refs[6:12]
    ubufs = refs[12:18]
    acc_sh = refs[18]
    gsems = refs[19:22]
    csems = refs[22:25]
    isems = refs[25:31]
    usems = refs[31:37]

    cid = lax.axis_index("c")
    sid = lax.axis_index("s")
    wid = sid * NC + cid

    @pl.when(sid < FLUSH_TILES)
    def _():
        zero = jnp.zeros((LANES,), jnp.float32)

        def zrow(g, _):
            rbufs[0][g // LG, pl.ds((g % LG) * LANES, LANES)] = zero
            return 0

        lax.fori_loop(0, MC * LG, zrow, 0)
        for t in range(FLUSH_ROWS // MC):
            pltpu.sync_copy(
                rbufs[0], acc_sh.at[pl.ds(sid * FLUSH_ROWS + t * MC, MC)]
            )

    plsc.subcore_barrier()

    def make_pipeline(nch, ibase, idx_hbm, idx_bufs, idx_sems,
                      tab_hbm, with_c):
        def fetch(g, r):
            pltpu.async_copy(
                idx_hbm.at[pl.ds(ibase + g, 1)], idx_bufs[r], idx_sems[r]
            )

        def gather(g, r, b):
            pltpu.make_async_copy(
                idx_hbm.at[pl.ds(ibase, 1)], idx_bufs[r], idx_sems[r]
            ).wait()
            pltpu.async_copy(
                tab_hbm.at[idx_bufs[r].at[0, 0]], rbufs[b], gsems[b]
            )
            if with_c:
                pltpu.async_copy(
                    c_hbm.at[pl.ds(wid * EPW + g * MC, MC)],
                    cbufs[b], csems[b],
                )

        def wait_rows(b):
            pltpu.make_async_copy(
                tab_hbm.at[idx_bufs[0].at[0, 0]], rbufs[b], gsems[b]
            ).wait()
            if with_c:
                pltpu.make_async_copy(
                    c_hbm.at[pl.ds(0, MC)], cbufs[b], csems[b]
                ).wait()

        def scatter(r, b):
            pltpu.sync_copy(
                rbufs[b], acc_sh.at[idx_bufs[r].at[0, 1]], add=True
            )

        return fetch, gather, wait_rows, scatter

    def run_pass(nch, ibase, idx_hbm, idx_bufs, idx_sems, tab_hbm, with_c,
                 compute):
        fetch, gather, wait_rows, scatter = make_pipeline(
            nch, ibase, idx_hbm, idx_bufs, idx_sems, tab_hbm, with_c
        )

        for g in range(min(NIB, nch)):
            fetch(g, g % NIB)
        for g in range(min(NBUF, nch)):
            gather(g, g % NIB, g % NBUF)

        def step(g, r, b, r_nxt, guard):
            wait_rows(b)
            if compute is not None:
                compute(b)
            scatter(r, b)
            nxt = g + NBUF
            nxt2 = g + NIB

            def advance():
                gather(nxt, r_nxt, b)

            def refetch():
                fetch(nxt2, r)

            if guard:
                if nxt < nch:
                    advance()
                if nxt2 < nch:
                    refetch()
            else:
                @pl.when(nxt < nch)
                def _():
                    advance()

                @pl.when(nxt2 < nch)
                def _():
                    refetch()

        def body(s, _):
            for k in range(NIB):
                g = s * NIB + k
                step(g, k, k % NBUF, (k + NBUF) % NIB, guard=False)
            return 0

        lax.fori_loop(0, nch // NIB, body, 0)
        for g in range(nch - nch % NIB, nch):
            step(g, g % NIB, g % NBUF, (g + NBUF) % NIB, guard=True)

    def compute_a(b):
        def row(r, _):
            for j in range(LG):
                col = j * LANES
                v = rbufs[b][r, pl.ds(col, LANES)] \
                    + cbufs[b][r, pl.ds(col, LANES)]
                rbufs[b][r, pl.ds(col, LANES)] = jnp.maximum(v, 0.0)
            return 0

        lax.fori_loop(0, MC, row, 0)

    run_pass(MCH, wid * MCH, sdr, ibufs, isems, zw_hbm, True, compute_a)

    run_pass(UCH, wid * UCH, usdr, ubufs, usems, zu_hbm, False, None)

    plsc.subcore_barrier()

    @pl.when(sid < FLUSH_TILES)
    def _():
        base = sid * FLUSH_ROWS
        pltpu.sync_copy(
            acc_sh.at[pl.ds(base, FLUSH_ROWS)],
            out_hbm.at[cid, pl.ds(base, FLUSH_ROWS)],
        )


@functools.cache
def _sc_layer():
    sems = [pltpu.SemaphoreType.DMA] * (3 + 3 + NIB + NIB)
    return pl.kernel(
        _sc_layer_kernel_body,
        out_type=jax.ShapeDtypeStruct((NC, N, H), jnp.float32),
        mesh=_mesh(),
        scratch_types=(
            [pltpu.VMEM((MC, H), jnp.float32)] * 3
            + [pltpu.VMEM((MC, H), jnp.float32)] * 3
            + [pltpu.VMEM((1, 2, MC), jnp.int32)] * NIB
            + [pltpu.VMEM((1, 2, MC), jnp.int32)] * NIB
            + [pltpu.VMEM_SHARED((N, H), jnp.float32)]
            + sems
        ),
    )


_EB = 2000
_NB = 2000


_PI_HI = 3.140625
_PI_LO = 9.67653589793e-4
_INV_PI = 0.3183098861837907
_S1 = -1.6666667163e-01
_S2 = 8.3333337680e-03
_S3 = -1.9841270114e-04
_S4 = 2.7557314297e-06


def _fast_sin(u):
    k = jnp.floor(u * _INV_PI + 0.5)
    x = u - k * _PI_HI - k * _PI_LO
    x2 = x * x
    p = x * (1.0 + x2 * (_S1 + x2 * (_S2 + x2 * (_S3 + x2 * _S4))))
    odd = (k.astype(jnp.int32) & 1) == 1
    return jnp.where(odd, -p, p)


def _edge_const_body(tsl_ref, td_ref, s0_ref, tew_ref, teb_ref,
                     wt_ref, bm_ref, c_ref):
    t = tsl_ref[0, 0, :]
    lin = (s0_ref[0, 0] * t + s0_ref[0, 1])[:, None]
    per = _fast_sin(
        t[:, None] * tew_ref[0, :][None, :] + teb_ref[0, :][None, :]
    )
    feats = jnp.concatenate([lin, per, td_ref[0, 0, :][:, None]], axis=1)
    c_ref[...] = (
        jnp.dot(feats, wt_ref[...], preferred_element_type=jnp.float32)
        + bm_ref[0, :][None, :]
    )


def _edge_const(tsl, td, te_w0, te_b0, te_w, te_b, Wt, bm):
    grid = E // _EB
    s0 = jnp.stack([te_w0, te_b0]).reshape(1, 2)
    return pl.pallas_call(
        _edge_const_body,
        grid=(grid,),
        in_specs=[
            pl.BlockSpec((1, 1, _EB), lambda i: (i, 0, 0)),
            pl.BlockSpec((1, 1, _EB), lambda i: (i, 0, 0)),
            pl.BlockSpec((1, 2), lambda i: (0, 0)),
            pl.BlockSpec((1, TF - 1), lambda i: (0, 0)),
            pl.BlockSpec((1, TF - 1), lambda i: (0, 0)),
            pl.BlockSpec((TF + 1, H), lambda i: (0, 0)),
            pl.BlockSpec((1, H), lambda i: (0, 0)),
        ],
        out_specs=pl.BlockSpec((_EB, H), lambda i: (i, 0)),
        out_shape=jax.ShapeDtypeStruct((E, H), jnp.float32),
    )(tsl.reshape(grid, 1, _EB), td.reshape(grid, 1, _EB), s0,
      te_w.reshape(1, TF - 1), te_b.reshape(1, TF - 1), Wt,
      bm.reshape(1, H))


def _pre_nodes_body(x_ref, wm_ref, wu_ref, zw_ref, zu_ref):
    xb = x_ref[...]
    zw_ref[...] = jnp.dot(xb, wm_ref[...], preferred_element_type=jnp.float32)
    zu_ref[...] = jnp.dot(xb, wu_ref[...], preferred_element_type=jnp.float32)


def _pre_nodes(xm, Wmx, Wu):
    return pl.pallas_call(
        _pre_nodes_body,
        grid=(N // _NB,),
        in_specs=[
            pl.BlockSpec((_NB, D), lambda i: (i, 0)),
            pl.BlockSpec((D, H), lambda i: (0, 0)),
            pl.BlockSpec((D, H), lambda i: (0, 0)),
        ],
        out_specs=[
            pl.BlockSpec((_NB, H), lambda i: (i, 0)),
            pl.BlockSpec((_NB, H), lambda i: (i, 0)),
        ],
        out_shape=[
            jax.ShapeDtypeStruct((N, H), jnp.float32),
            jax.ShapeDtypeStruct((N, H), jnp.float32),
        ],
    )(xm, Wmx, Wu)


def _mid_body(acc_ref, z_ref, ws_ref, bo_ref, wm_ref, wu_ref,
              z1_ref, zw_ref, zu_ref):
    a = acc_ref[0] + acc_ref[1]
    a += jnp.dot(z_ref[...], ws_ref[...], preferred_element_type=jnp.float32)
    z1 = jnp.maximum(a + bo_ref[0, :][None, :], 0.0)
    z1_ref[...] = z1
    zw_ref[...] = jnp.dot(z1, wm_ref[...], preferred_element_type=jnp.float32)
    zu_ref[...] = jnp.dot(z1, wu_ref[...], preferred_element_type=jnp.float32)


def _mid(acc, z, Ws, bo, Wmx, Wu):
    return pl.pallas_call(
        _mid_body,
        grid=(N // _NB,),
        in_specs=[
            pl.BlockSpec((NC, _NB, H), lambda i: (0, i, 0)),
            pl.BlockSpec((_NB, D), lambda i: (i, 0)),
            pl.BlockSpec((D, H), lambda i: (0, 0)),
            pl.BlockSpec((1, H), lambda i: (0, 0)),
            pl.BlockSpec((D, H), lambda i: (0, 0)),
            pl.BlockSpec((D, H), lambda i: (0, 0)),
        ],
        out_specs=[
            pl.BlockSpec((_NB, H), lambda i: (i, 0)),
            pl.BlockSpec((_NB, H), lambda i: (i, 0)),
            pl.BlockSpec((_NB, H), lambda i: (i, 0)),
        ],
        out_shape=[
            jax.ShapeDtypeStruct((N, H), jnp.float32),
            jax.ShapeDtypeStruct((N, H), jnp.float32),
            jax.ShapeDtypeStruct((N, H), jnp.float32),
        ],
    )(acc, z, Ws, bo.reshape(1, H), Wmx, Wu)


def _final_body(acc_ref, z_ref, ws_ref, bo_ref, o_ref):
    a = acc_ref[0] + acc_ref[1]
    a += jnp.dot(z_ref[...], ws_ref[...], preferred_element_type=jnp.float32)
    o_ref[...] = jnp.maximum(a + bo_ref[0, :][None, :], 0.0)


def _final(acc, z, Ws, bo):
    return pl.pallas_call(
        _final_body,
        grid=(N // _NB,),
        in_specs=[
            pl.BlockSpec((NC, _NB, H), lambda i: (0, i, 0)),
            pl.BlockSpec((_NB, D), lambda i: (i, 0)),
            pl.BlockSpec((D, H), lambda i: (0, 0)),
            pl.BlockSpec((1, H), lambda i: (0, 0)),
        ],
        out_specs=pl.BlockSpec((_NB, H), lambda i: (i, 0)),
        out_shape=jax.ShapeDtypeStruct((N, H), jnp.float32),
    )(acc, z, Ws, bo.reshape(1, H))


def kernel(x, edge_index, timestamps, time_diffs, unique_edges,
           timestamp_lists, te_w0, te_b0, te_w, te_b,
           Wm0, bm0, Wu0, Ws0, bo0, Wm1, bm1, Wu1, Ws1, bo1):
    sdr = jnp.stack(
        [edge_index[0].reshape(NW * MCH, MC),
         edge_index[1].reshape(NW * MCH, MC)], axis=1)
    usdr = jnp.stack(
        [unique_edges[0].reshape(NW * UCH, MC),
         unique_edges[1].reshape(NW * UCH, MC)], axis=1)

    c0 = _edge_const(timestamp_lists, time_diffs, te_w0, te_b0,
                     te_w, te_b, Wm0[D:], bm0)
    zw0, zu0 = _pre_nodes(x, Wm0[:D], Wu0)
    acc0 = _sc_layer()(zw0, zu0, c0, sdr, usdr)
    c1 = _edge_const(timestamp_lists, time_diffs, te_w0, te_b0,
                     te_w, te_b, Wm1[D:], bm1)
    z1, zw1, zu1 = _mid(acc0, x, Ws0, bo0, Wm1[:D], Wu1)
    acc1 = _sc_layer()(zw1, zu1, c1, sdr, usdr)
    return _final(acc1, z1, Ws1, bo1)
